# baseline (speedup 1.0000x reference)
import functools
import jax
import jax.numpy as jnp
from jax import lax
from jax.experimental import pallas as pl
from jax.experimental.pallas import tpu as pltpu
from jax.experimental.pallas import tpu_sc as plsc

_NW = 32
_G = 5000
_T = 157
_LGW = 16 * 91
_BXW = 16 * 4


def _sc_post(lg_hbm, bx_hbm, cls_hbm, ts_hbm,
             sc_out, lb_out, bx_out, cl_out,
             lgA, lgB, bxA, bxB,
             osA, osB, olA, olB, obA, obB,
             tsv, clsv, cll,
             semA, semB, osemA, osemB):
    wid = lax.axis_index("s") * 2 + lax.axis_index("c")
    iota = lax.iota(jnp.int32, 16)
    base91 = iota * 91
    base4 = iota * 4

    @pl.when(wid == 0)
    def _():
        pltpu.sync_copy(cls_hbm, clsv)
        m0 = plsc.load_gather(clsv, [iota * 10])
        lab0 = jnp.zeros((16,), jnp.int32)

        def cbody(c, carry):
            m, lab = carry
            v = plsc.load_gather(clsv, [iota * 10 + c])
            upd = v > m
            return jnp.where(upd, v, m), jnp.where(upd, c, lab)

        _, lab = lax.fori_loop(1, 10, cbody, (m0, lab0))
        cll[...] = lab
        pltpu.sync_copy(cll, cl_out)

    pltpu.sync_copy(ts_hbm, tsv)

    def start_in(g, lg_buf, bx_buf, sem):
        pltpu.async_copy(lg_hbm.at[pl.ds(g * _LGW, _LGW)], lg_buf, sem)
        pltpu.async_copy(bx_hbm.at[pl.ds(g * _BXW, _BXW)], bx_buf, sem)

    def wait_in(lg_buf, bx_buf, sem):
        pltpu.make_async_copy(lg_hbm.at[pl.ds(0, _LGW)], lg_buf, sem).wait()
        pltpu.make_async_copy(bx_hbm.at[pl.ds(0, _BXW)], bx_buf, sem).wait()

    def wait_out(os_, ol_, ob_, osem):
        pltpu.make_async_copy(lg_hbm.at[pl.ds(0, 16)], os_, osem).wait()
        pltpu.make_async_copy(lg_hbm.at[pl.ds(0, 16)], ol_, osem).wait()
        pltpu.make_async_copy(lg_hbm.at[pl.ds(0, 64)], ob_, osem).wait()

    start_in(wid, lgA, bxA, semA)
    start_in(wid + _NW, lgB, bxB, semB)

    def slot(i, b, lg_buf, bx_buf, os_, ol_, ob_, sem, osem):
        tt = 2 * i + b
        g = wid + _NW * tt

        @pl.when(g < _G)
        def _():
            wait_in(lg_buf, bx_buf, sem)

            @pl.when(tt >= 2)
            def _():
                wait_out(os_, ol_, ob_, osem)

            m0 = plsc.load_gather(lg_buf, [base91])
            s0 = jnp.exp(m0)
            lab0 = jnp.zeros((16,), jnp.int32)

            def body(c, carry):
                m, lab, s = carry
                v = plsc.load_gather(lg_buf, [base91 + c])
                upd = v > m
                return (jnp.where(upd, v, m), jnp.where(upd, c, lab),
                        s + jnp.exp(v))

            m, lab, s = lax.fori_loop(1, 91, body, (m0, lab0, s0),
                                      unroll=7)
            os_[...] = jnp.exp(m) / s
            ol_[...] = lab

            row = g * 16 + iota
            b2 = (row // _G) * 2
            sh = plsc.load_gather(tsv, [b2]).astype(jnp.float32)
            sw = plsc.load_gather(tsv, [b2 + 1]).astype(jnp.float32)
            cx = plsc.load_gather(bx_buf, [base4])
            cy = plsc.load_gather(bx_buf, [base4 + 1])
            w = plsc.load_gather(bx_buf, [base4 + 2])
            h = plsc.load_gather(bx_buf, [base4 + 3])
            one = jnp.float32(1.0)
            zero = jnp.float32(0.0)
            x0 = jnp.clip(cx - 0.5 * w, zero, one) * sw
            y0 = jnp.clip(cy - 0.5 * h, zero, one) * sh
            x1 = jnp.clip(cx + 0.5 * w, zero, one) * sw
            y1 = jnp.clip(cy + 0.5 * h, zero, one) * sh
            plsc.store_scatter(ob_, [base4], x0)
            plsc.store_scatter(ob_, [base4 + 1], y0)
            plsc.store_scatter(ob_, [base4 + 2], x1)
            plsc.store_scatter(ob_, [base4 + 3], y1)

            pltpu.async_copy(os_, sc_out.at[pl.ds(g * 16, 16)], osem)
            pltpu.async_copy(ol_, lb_out.at[pl.ds(g * 16, 16)], osem)
            pltpu.async_copy(ob_, bx_out.at[pl.ds(g * 64, 64)], osem)

            g2 = g + 2 * _NW

            @pl.when(g2 < _G)
            def _():
                start_in(g2, lg_buf, bx_buf, sem)

    def lbody(i, _):
        slot(i, 0, lgA, bxA, osA, olA, obA, semA, osemA)
        slot(i, 1, lgB, bxB, osB, olB, obB, semB, osemB)
        return 0

    lax.fori_loop(0, (_T + 1) // 2, lbody, 0)

    wait_out(osA, olA, obA, osemA)
    wait_out(osB, olB, obB, osemB)


def kernel(pred_logits, pred_boxes, cls_logits, target_sizes):
    nb, nq, nc = pred_logits.shape
    mesh = plsc.VectorSubcoreMesh(core_axis_name="c", subcore_axis_name="s")
    fn = functools.partial(
        pl.kernel,
        mesh=mesh,
        compiler_params=pltpu.CompilerParams(needs_layout_passes=False),
        out_type=[
            jax.ShapeDtypeStruct((nb * nq,), jnp.float32),
            jax.ShapeDtypeStruct((nb * nq,), jnp.int32),
            jax.ShapeDtypeStruct((nb * nq * 4,), jnp.float32),
            jax.ShapeDtypeStruct((nb,), jnp.int32),
        ],
        scratch_types=[
            pltpu.VMEM((_LGW,), jnp.float32),
            pltpu.VMEM((_LGW,), jnp.float32),
            pltpu.VMEM((_BXW,), jnp.float32),
            pltpu.VMEM((_BXW,), jnp.float32),
            pltpu.VMEM((16,), jnp.float32),
            pltpu.VMEM((16,), jnp.float32),
            pltpu.VMEM((16,), jnp.int32),
            pltpu.VMEM((16,), jnp.int32),
            pltpu.VMEM((64,), jnp.float32),
            pltpu.VMEM((64,), jnp.float32),
            pltpu.VMEM((32,), jnp.int32),
            pltpu.VMEM((160,), jnp.float32),
            pltpu.VMEM((16,), jnp.int32),
            pltpu.SemaphoreType.DMA,
            pltpu.SemaphoreType.DMA,
            pltpu.SemaphoreType.DMA,
            pltpu.SemaphoreType.DMA,
        ],
    )(_sc_post)
    scores, labels, boxes, cls2 = fn(
        pred_logits.reshape(-1), pred_boxes.reshape(-1),
        cls_logits.reshape(-1), target_sizes.reshape(-1))
    return (scores.reshape(nb, nq), labels.reshape(nb, nq),
            boxes.reshape(nb, nq, 4), cls2)


# speedup vs baseline: 1.1355x; 1.1355x over previous
import functools
import jax
import jax.numpy as jnp
from jax import lax
from jax.experimental import pallas as pl
from jax.experimental.pallas import tpu as pltpu
from jax.experimental.pallas import tpu_sc as plsc

_NW = 32
_GB = 313
_G = 16 * _GB
_T = (_G + _NW - 1) // _NW


def _sc_post(lg_hbm, bx_hbm, cls_hbm, ts_hbm,
             sc_out, lb_out, bx_out, cl_out,
             lgA, lgB, bxA, bxB,
             osA, osB, olA, olB, obA, obB,
             tsv, clsv, cll,
             semA, semB, osemA, osemB):
    wid = lax.axis_index("s") * 2 + lax.axis_index("c")
    iota = lax.iota(jnp.int32, 16)
    base4 = iota * 4

    @pl.when(wid == 0)
    def _():
        pltpu.sync_copy(cls_hbm, clsv)
        m0 = plsc.load_gather(clsv, [iota * 10])
        lab0 = jnp.zeros((16,), jnp.int32)

        def cbody(c, carry):
            m, lab = carry
            v = plsc.load_gather(clsv, [iota * 10 + c])
            upd = v > m
            return jnp.where(upd, v, m), jnp.where(upd, c, lab)

        _, lab = lax.fori_loop(1, 10, cbody, (m0, lab0))
        cll[...] = lab
        pltpu.sync_copy(cll, cl_out)

    pltpu.sync_copy(ts_hbm, tsv)

    def g_to_br(g):
        b = g // _GB
        t = g - b * _GB
        r = jnp.where(t == _GB - 1, 5000 - 16, t * 16)
        return b, pl.multiple_of(r, 8)

    def start_in(g, lg_buf, bx_buf, sem):
        b, r = g_to_br(g)
        pltpu.async_copy(lg_hbm.at[b, pl.ds(r, 16), :], lg_buf, sem)
        pltpu.async_copy(bx_hbm.at[pl.ds((b * 5000 + r) * 4, 64)], bx_buf,
                         sem)

    def wait_in(lg_buf, bx_buf, sem):
        pltpu.make_async_copy(lg_hbm.at[0, pl.ds(0, 16), :], lg_buf,
                              sem).wait()
        pltpu.make_async_copy(bx_hbm.at[pl.ds(0, 64)], bx_buf, sem).wait()

    def wait_out(os_, ol_, ob_, osem):
        pltpu.make_async_copy(sc_out.at[pl.ds(0, 16)], os_, osem).wait()
        pltpu.make_async_copy(lb_out.at[pl.ds(0, 16)], ol_, osem).wait()
        pltpu.make_async_copy(bx_out.at[pl.ds(0, 64)], ob_, osem).wait()

    start_in(wid, lgA, bxA, semA)
    start_in(wid + _NW, lgB, bxB, semB)

    def slot(i, sl, lg_buf, bx_buf, os_, ol_, ob_, sem, osem):
        tt = 2 * i + sl
        g = wid + _NW * tt

        @pl.when(g < _G)
        def _():
            b, r = g_to_br(g)
            row0 = b * 5000 + r
            wait_in(lg_buf, bx_buf, sem)

            @pl.when(tt >= 2)
            def _():
                wait_out(os_, ol_, ob_, osem)

            zero = jnp.zeros((16,), jnp.int32)
            m0 = plsc.load_gather(lg_buf, [iota, zero])
            s0 = jnp.exp(m0)
            lab0 = zero

            def body(c, carry):
                m, lab, s = carry
                v = plsc.load_gather(lg_buf, [iota, zero + c])
                upd = v > m
                return (jnp.where(upd, v, m), jnp.where(upd, c, lab),
                        s + jnp.exp(v))

            m, lab, s = lax.fori_loop(1, 91, body, (m0, lab0, s0),
                                      unroll=7)
            os_[...] = jnp.exp(m) / s
            ol_[...] = lab

            shv = plsc.load_gather(tsv, [zero + 2 * b]).astype(jnp.float32)
            swv = plsc.load_gather(tsv, [zero + 2 * b + 1]).astype(
                jnp.float32)
            cx = plsc.load_gather(bx_buf, [base4])
            cy = plsc.load_gather(bx_buf, [base4 + 1])
            w = plsc.load_gather(bx_buf, [base4 + 2])
            h = plsc.load_gather(bx_buf, [base4 + 3])
            one = jnp.float32(1.0)
            zf = jnp.float32(0.0)
            x0 = jnp.clip(cx - 0.5 * w, zf, one) * swv
            y0 = jnp.clip(cy - 0.5 * h, zf, one) * shv
            x1 = jnp.clip(cx + 0.5 * w, zf, one) * swv
            y1 = jnp.clip(cy + 0.5 * h, zf, one) * shv
            plsc.store_scatter(ob_, [base4], x0)
            plsc.store_scatter(ob_, [base4 + 1], y0)
            plsc.store_scatter(ob_, [base4 + 2], x1)
            plsc.store_scatter(ob_, [base4 + 3], y1)

            pltpu.async_copy(os_, sc_out.at[pl.ds(row0, 16)], osem)
            pltpu.async_copy(ol_, lb_out.at[pl.ds(row0, 16)], osem)
            pltpu.async_copy(ob_, bx_out.at[pl.ds(row0 * 4, 64)], osem)

            g2 = g + 2 * _NW

            @pl.when(g2 < _G)
            def _():
                start_in(g2, lg_buf, bx_buf, sem)

    def lbody(i, _):
        slot(i, 0, lgA, bxA, osA, olA, obA, semA, osemA)
        slot(i, 1, lgB, bxB, osB, olB, obB, semB, osemB)
        return 0

    lax.fori_loop(0, (_T + 1) // 2, lbody, 0)

    wait_out(osA, olA, obA, osemA)
    wait_out(osB, olB, obB, osemB)


def kernel(pred_logits, pred_boxes, cls_logits, target_sizes):
    nb, nq, nc = pred_logits.shape
    mesh = plsc.VectorSubcoreMesh(core_axis_name="c", subcore_axis_name="s")
    fn = functools.partial(
        pl.kernel,
        mesh=mesh,
        compiler_params=pltpu.CompilerParams(needs_layout_passes=False),
        out_type=[
            jax.ShapeDtypeStruct((nb * nq,), jnp.float32),
            jax.ShapeDtypeStruct((nb * nq,), jnp.int32),
            jax.ShapeDtypeStruct((nb * nq * 4,), jnp.float32),
            jax.ShapeDtypeStruct((nb,), jnp.int32),
        ],
        scratch_types=[
            pltpu.VMEM((16, 91), jnp.float32),
            pltpu.VMEM((16, 91), jnp.float32),
            pltpu.VMEM((64,), jnp.float32),
            pltpu.VMEM((64,), jnp.float32),
            pltpu.VMEM((16,), jnp.float32),
            pltpu.VMEM((16,), jnp.float32),
            pltpu.VMEM((16,), jnp.int32),
            pltpu.VMEM((16,), jnp.int32),
            pltpu.VMEM((64,), jnp.float32),
            pltpu.VMEM((64,), jnp.float32),
            pltpu.VMEM((32,), jnp.int32),
            pltpu.VMEM((160,), jnp.float32),
            pltpu.VMEM((16,), jnp.int32),
            pltpu.SemaphoreType.DMA,
            pltpu.SemaphoreType.DMA,
            pltpu.SemaphoreType.DMA,
            pltpu.SemaphoreType.DMA,
        ],
    )(_sc_post)
    scores, labels, boxes, cls2 = fn(
        pred_logits, pred_boxes.reshape(-1), cls_logits.reshape(-1),
        target_sizes.reshape(-1))
    return (scores.reshape(nb, nq), labels.reshape(nb, nq),
            boxes.reshape(nb, nq, 4), cls2)
